# speedup vs baseline: 2.9710x; 2.9710x over previous
import functools

import jax
import jax.numpy as jnp
from jax import lax
from jax.experimental import pallas as pl
from jax.experimental.pallas import tpu as pltpu
from jax.experimental.pallas import tpu_sc as plsc

_D = 128
_NC = 2
_NS = 16
_NW = _NC * _NS
_CHUNK = 128


@functools.lru_cache(maxsize=None)
def _make_gather(B, V):
    b_per_w = B // _NW
    nchunk = b_per_w // _CHUNK
    mesh = plsc.VectorSubcoreMesh(core_axis_name="c", subcore_axis_name="s")

    @functools.partial(
        pl.kernel,
        mesh=mesh,
        out_type=jax.ShapeDtypeStruct((B, _D), jnp.float32),
        scratch_types=[
            pltpu.VMEM((nchunk, _CHUNK), jnp.int32),
            pltpu.VMEM((_CHUNK, _D), jnp.float32),
            pltpu.SemaphoreType.DMA,
        ],
    )
    def gather_kernel(idx_hbm, table_hbm, out_hbm, idx_v, buf, sem):
        wid = lax.axis_index("s") * _NC + lax.axis_index("c")
        base = wid * b_per_w
        pltpu.sync_copy(idx_hbm.at[wid], idx_v)

        def body(c, carry):
            pltpu.async_copy(table_hbm.at[idx_v.at[c]], buf, sem).wait()
            pltpu.sync_copy(buf, out_hbm.at[pl.ds(base + c * _CHUNK, _CHUNK)])
            return carry

        lax.fori_loop(0, nchunk, body, 0)

    return gather_kernel


def kernel(resids_positional_encoded, table):
    idx = resids_positional_encoded.astype(jnp.int32)
    n, s = idx.shape
    B = n * s
    V, D = table.shape
    idx3 = idx.reshape(_NW, B // (_NW * _CHUNK), _CHUNK)
    out = _make_gather(B, V)(idx3, table)
    return out.reshape(n, s, D)


# speedup vs baseline: 3.3539x; 1.1289x over previous
import functools

import jax
import jax.numpy as jnp
from jax import lax
from jax.experimental import pallas as pl
from jax.experimental.pallas import tpu as pltpu
from jax.experimental.pallas import tpu_sc as plsc

_D = 128
_NC = 2
_NS = 16
_NW = _NC * _NS
_CHUNK = 128
_NBUF = 2


@functools.lru_cache(maxsize=None)
def _make_gather(B, V):
    b_per_w = B // _NW
    nchunk = b_per_w // _CHUNK
    mesh = plsc.VectorSubcoreMesh(core_axis_name="c", subcore_axis_name="s")

    @functools.partial(
        pl.kernel,
        mesh=mesh,
        out_type=jax.ShapeDtypeStruct((B, _D), jnp.float32),
        scratch_types=[
            pltpu.VMEM((nchunk, _CHUNK), jnp.int32),
            pltpu.VMEM((_NBUF, _CHUNK, _D), jnp.float32),
            pltpu.SemaphoreType.DMA,
            pltpu.SemaphoreType.DMA,
        ],
    )
    def gather_kernel(idx_hbm, table_hbm, out_hbm, idx_v, bufs, gsem, wsem):
        wid = lax.axis_index("s") * _NC + lax.axis_index("c")
        base = wid * b_per_w
        pltpu.sync_copy(idx_hbm.at[wid], idx_v)

        def gather_copy(c):
            return pltpu.make_async_copy(
                table_hbm.at[idx_v.at[c]], bufs.at[c % _NBUF], gsem)

        def write_copy(c):
            return pltpu.make_async_copy(
                bufs.at[c % _NBUF],
                out_hbm.at[pl.ds(base + c * _CHUNK, _CHUNK)], wsem)

        for c in range(_NBUF):
            gather_copy(c).start()

        def body(c, carry):
            gather_copy(c).wait()
            write_copy(c).start()

            @pl.when(c + _NBUF < nchunk)
            def _():
                write_copy(c).wait()
                gather_copy(c + _NBUF).start()

            return carry

        lax.fori_loop(0, nchunk, body, 0)
        for _ in range(_NBUF):
            write_copy(0).wait()

    return gather_kernel


def kernel(resids_positional_encoded, table):
    idx = resids_positional_encoded.astype(jnp.int32)
    n, s = idx.shape
    B = n * s
    V, D = table.shape
    idx3 = idx.reshape(_NW, B // (_NW * _CHUNK), _CHUNK)
    out = _make_gather(B, V)(idx3, table)
    return out.reshape(n, s, D)


# speedup vs baseline: 3.3542x; 1.0001x over previous
import functools

import jax
import jax.numpy as jnp
from jax import lax
from jax.experimental import pallas as pl
from jax.experimental.pallas import tpu as pltpu
from jax.experimental.pallas import tpu_sc as plsc

_D = 128
_NC = 2
_NS = 16
_NW = _NC * _NS
_CHUNK = 128
_NBUF = 4


@functools.lru_cache(maxsize=None)
def _make_gather(B, V):
    b_per_w = B // _NW
    nchunk = b_per_w // _CHUNK
    mesh = plsc.VectorSubcoreMesh(core_axis_name="c", subcore_axis_name="s")

    @functools.partial(
        pl.kernel,
        mesh=mesh,
        out_type=jax.ShapeDtypeStruct((B, _D), jnp.float32),
        scratch_types=[
            pltpu.VMEM((nchunk, _CHUNK), jnp.int32),
            pltpu.VMEM((_NBUF, _CHUNK, _D), jnp.float32),
            pltpu.SemaphoreType.DMA,
            pltpu.SemaphoreType.DMA,
        ],
    )
    def gather_kernel(idx_hbm, table_hbm, out_hbm, idx_v, bufs, gsem, wsem):
        wid = lax.axis_index("s") * _NC + lax.axis_index("c")
        base = wid * b_per_w
        pltpu.sync_copy(idx_hbm.at[wid], idx_v)

        def gather_copy(c):
            return pltpu.make_async_copy(
                table_hbm.at[idx_v.at[c]], bufs.at[c % _NBUF], gsem)

        def write_copy(c):
            return pltpu.make_async_copy(
                bufs.at[c % _NBUF],
                out_hbm.at[pl.ds(base + c * _CHUNK, _CHUNK)], wsem)

        for c in range(_NBUF):
            gather_copy(c).start()

        def body(c, carry):
            gather_copy(c).wait()
            write_copy(c).start()

            @pl.when(c + _NBUF < nchunk)
            def _():
                write_copy(c).wait()
                gather_copy(c + _NBUF).start()

            return carry

        lax.fori_loop(0, nchunk, body, 0)
        for _ in range(_NBUF):
            write_copy(0).wait()

    return gather_kernel


def kernel(resids_positional_encoded, table):
    idx = resids_positional_encoded.astype(jnp.int32)
    n, s = idx.shape
    B = n * s
    V, D = table.shape
    idx3 = idx.reshape(_NW, B // (_NW * _CHUNK), _CHUNK)
    out = _make_gather(B, V)(idx3, table)
    return out.reshape(n, s, D)
